# baseline (speedup 1.0000x reference)
import functools

import jax
import jax.numpy as jnp
from jax import lax
from jax.experimental import pallas as pl
from jax.experimental.pallas import tpu as pltpu
from jax.experimental.pallas import tpu_sc as plsc

N = 10000
E = 160000
NC, NS = 2, 16
NP = N
EP = E
B = 100
RPT = NP // NS
EPT = EP // NS
DEGC = 16

f32 = jnp.float32
i32 = jnp.int32


@functools.lru_cache(maxsize=None)
def _make_sc_agg(C, edge_split=False):
    P = 1 if edge_split else C // NC
    n_out = NC if edge_split else C
    mesh = plsc.VectorSubcoreMesh(
        core_axis_name="c", subcore_axis_name="s",
        num_cores=NC, num_subcores=NS)
    if edge_split:
        bat, SL, DEPTH = 100, 50, 3
    else:
        bat, SL, DEPTH = 80, 25, 4
    G = DEPTH - 1
    nbatch = (EPT // NC if edge_split else EPT) // bat
    NSTG = nbatch // SL
    outs = [jax.ShapeDtypeStruct((NP, 128), f32) for _ in range(n_out)]
    scratch = [
        pltpu.VMEM_SHARED((NP, 128), f32),
        pltpu.VMEM((SL, bat), i32),
        pltpu.VMEM((SL, bat), i32),
    ] + [pltpu.VMEM((bat, 128), f32) for _ in range(DEPTH)] \
      + [pltpu.SemaphoreType.DMA] * (2 * DEPTH)

    def body(*refs):
        xs = refs[:C]
        srcr, dstr, zrow = refs[C:C + 3]
        outs_r = refs[C + 3:C + 3 + n_out]
        rest = refs[C + 3 + n_out:]
        acc, srcall, dstall = rest[:3]
        rows = rest[3:3 + DEPTH]
        semG = rest[3 + DEPTH:3 + 2 * DEPTH]
        semS = rest[3 + 2 * DEPTH:3 + 3 * DEPTH]

        cid = lax.axis_index("c")
        sid = lax.axis_index("s")
        rs = pl.ds(sid * RPT, RPT)

        for p in range(P):
            pltpu.sync_copy(zrow, acc.at[rs])
            plsc.subcore_barrier()
            for c in range(NC):
                @pl.when(cid == c)
                def _(p=p, c=c):
                    if edge_split:
                        chunk, slot = 0, c
                        row0 = (c * NS + sid) * nbatch
                    else:
                        chunk = slot = c * P + p
                        row0 = sid * nbatch
                    x = xs[chunk]

                    def start_g(i, k):
                        pltpu.async_copy(x.at[srcall.at[i]], rows[k],
                                         semG[k])

                    def wait_g(k):
                        pltpu.make_async_copy(x.at[srcall.at[0]], rows[k],
                                              semG[k]).wait()

                    def start_s(i, k):
                        pltpu.async_copy(rows[k], acc.at[dstall.at[i]],
                                         semS[k], add=True)

                    def wait_s(k):
                        pltpu.make_async_copy(rows[k], acc.at[dstall.at[0]],
                                              semS[k]).wait()

                    def stage(s, carry):
                        pltpu.sync_copy(
                            srcr.at[pl.ds(row0 + s * SL, SL)], srcall)
                        pltpu.sync_copy(
                            dstr.at[pl.ds(row0 + s * SL, SL)], dstall)
                        for j in range(G):
                            start_g(j, j)
                        for i in range(SL):
                            wait_g(i % DEPTH)
                            start_s(i, i % DEPTH)
                            if i + G < SL:
                                kn = (i + G) % DEPTH
                                if i >= 1:
                                    wait_s(kn)
                                start_g(i + G, kn)
                        for j in range(SL - G - 1, SL):
                            wait_s(j % DEPTH)
                        return carry

                    lax.fori_loop(0, NSTG, stage, 0)
            plsc.subcore_barrier()
            for c in range(NC):
                @pl.when(cid == c)
                def _(p=p, c=c):
                    slot = c if edge_split else c * P + p
                    pltpu.sync_copy(acc.at[rs], outs_r[slot].at[rs])

    return pl.kernel(body, out_type=outs, mesh=mesh, scratch_types=scratch,
                     compiler_params=pltpu.CompilerParams(
                         use_tc_tiling_on_sc=False),
                     name=f"sc_agg_c{C}" + ("_es" if edge_split else ""))


@functools.lru_cache(maxsize=None)
def _make_sc_deg():
    mesh = plsc.VectorSubcoreMesh(
        core_axis_name="c", subcore_axis_name="s",
        num_cores=NC, num_subcores=NS)
    nbatch = EPT // NC // B
    outs = [jax.ShapeDtypeStruct((NP, DEGC), f32) for _ in range(NC)]
    scratch = [
        pltpu.VMEM_SHARED((NP, DEGC), f32),
        pltpu.VMEM((nbatch, B), i32),
        pltpu.VMEM((B, DEGC), f32),
    ]

    def body(dstr, z16, ones_h, out0, out1, dacc, dstall, onesv):
        cid = lax.axis_index("c")
        sid = lax.axis_index("s")
        rs = pl.ds(sid * RPT, RPT)
        pltpu.sync_copy(z16, dacc.at[rs])
        pltpu.sync_copy(ones_h, onesv)
        for c in range(NC):
            @pl.when(cid == c)
            def _(c=c):
                row0 = (c * NS + sid) * nbatch
                pltpu.sync_copy(dstr.at[pl.ds(row0, nbatch)], dstall)
        plsc.subcore_barrier()

        def step(i, carry):
            pltpu.sync_copy(onesv, dacc.at[dstall.at[i]], add=True)
            return carry

        lax.fori_loop(0, nbatch, step, 0)
        plsc.subcore_barrier()
        outs_r = (out0, out1)
        for c in range(NC):
            @pl.when(cid == c)
            def _(c=c):
                pltpu.sync_copy(dacc.at[rs], outs_r[c].at[rs])

    return pl.kernel(body, out_type=outs, mesh=mesh, scratch_types=scratch,
                     compiler_params=pltpu.CompilerParams(
                         use_tc_tiling_on_sc=False),
                     name="sc_deg")


def _row_spec(BN):
    return pl.BlockSpec((BN, 128), lambda n: (n, 0))


def _make_tc_layer(C_in, C_out, relu, BN=200):
    def body(*refs):
        aggs = refs[:C_in]
        xs = refs[C_in:2 * C_in]
        inv, w, b = refs[2 * C_in:2 * C_in + 3]
        outs = refs[2 * C_in + 3:]
        h = jnp.concatenate(
            [(aggs[c][...] + xs[c][...]) * inv[...] for c in range(C_in)],
            axis=1)
        z = jnp.dot(h, w[...], preferred_element_type=f32) + b[...]
        if relu:
            z = jnp.maximum(z, 0.0)
        for co in range(C_out):
            outs[co][...] = z[:, co * 128:(co + 1) * 128]

    return pl.pallas_call(
        body,
        grid=(NP // BN,),
        in_specs=[_row_spec(BN)] * (2 * C_in) + [
            pl.BlockSpec((BN, 1), lambda n: (n, 0)),
            pl.BlockSpec((C_in * 128, C_out * 128), lambda n: (0, 0)),
            pl.BlockSpec((1, C_out * 128), lambda n: (0, 0)),
        ],
        out_specs=[_row_spec(BN)] * C_out,
        out_shape=[jax.ShapeDtypeStruct((NP, 128), f32)] * C_out,
    )


def _make_tc_layer_mm(C_in, C_out, BN=200):
    def body(*refs):
        aggs = refs[:C_in]
        xs = refs[C_in:2 * C_in]
        inv, w, b, w2 = refs[2 * C_in:2 * C_in + 4]
        outs = refs[2 * C_in + 4:]
        h = jnp.concatenate(
            [(aggs[c][...] + xs[c][...]) * inv[...] for c in range(C_in)],
            axis=1)
        z = jnp.maximum(
            jnp.dot(h, w[...], preferred_element_type=f32) + b[...], 0.0)
        outs[0][...] = jnp.dot(z, w2[...], preferred_element_type=f32)

    return pl.pallas_call(
        body,
        grid=(NP // BN,),
        in_specs=[_row_spec(BN)] * (2 * C_in) + [
            pl.BlockSpec((BN, 1), lambda n: (n, 0)),
            pl.BlockSpec((C_in * 128, C_out * 128), lambda n: (0, 0)),
            pl.BlockSpec((1, C_out * 128), lambda n: (0, 0)),
            pl.BlockSpec((C_out * 128, 128), lambda n: (0, 0)),
        ],
        out_specs=[_row_spec(BN)],
        out_shape=[jax.ShapeDtypeStruct((NP, 128), f32)],
    )


def _make_tc_combine(BN=200):
    def body(p0, p1, y, inv, b, out):
        out[...] = (p0[...] + p1[...] + y[...]) * inv[...] + b[...]

    return pl.pallas_call(
        body,
        grid=(NP // BN,),
        in_specs=[_row_spec(BN)] * 3 + [
            pl.BlockSpec((BN, 1), lambda n: (n, 0)),
            pl.BlockSpec((1, 128), lambda n: (0, 0)),
        ],
        out_specs=_row_spec(BN),
        out_shape=jax.ShapeDtypeStruct((NP, 128), f32),
    )


_tc_l0 = _make_tc_layer(2, 4, relu=True)
_tc_mid = _make_tc_layer(4, 4, relu=True)
_tc_l3mm = _make_tc_layer_mm(4, 4)
_tc_combine = _make_tc_combine()


def kernel(feat, edge_index, W0, b0, W1, b1, W2, b2, W3, b3, W4, b4):
    _sc_deg = _make_sc_deg()
    _sc_agg_l0 = _make_sc_agg(2)
    _sc_agg_mid = _make_sc_agg(4)
    _sc_agg_last = _make_sc_agg(1, edge_split=True)

    src = edge_index[0].astype(i32)
    dst = edge_index[1].astype(i32)
    srcp80 = src.reshape(EP // 80, 80)
    dstp80 = dst.reshape(EP // 80, 80)
    srcp = src.reshape(EP // B, B)
    dstp = dst.reshape(EP // B, B)

    xc = [feat[:, 0:128], feat[:, 128:256]]

    z128 = jnp.zeros((RPT, 128), f32)
    z16 = jnp.zeros((RPT, DEGC), f32)
    ones = jnp.ones((B, DEGC), f32)

    d0, d1 = _sc_deg(dstp, z16, ones)
    inv = (1.0 / (d0[:, 0] + d1[:, 0] + 1.0)).reshape(NP, 1)

    agg0 = _sc_agg_l0(*xc, srcp80, dstp80, z128)

    xc = _tc_l0(*agg0, *xc, inv, W0, b0.reshape(1, 512))
    for W, b in ((W1, b1), (W2, b2)):
        aggs = _sc_agg_mid(*xc, srcp80, dstp80, z128)
        xc = _tc_mid(*aggs, *xc, inv, W, b.reshape(1, 512))

    aggs = _sc_agg_mid(*xc, srcp80, dstp80, z128)
    (y,) = _tc_l3mm(*aggs, *xc, inv, W3, b3.reshape(1, 512), W4)
    p0, p1 = _sc_agg_last(y, srcp, dstp, z128)
    return _tc_combine(p0, p1, y, inv, b4.reshape(1, 128))


# speedup vs baseline: 1.0703x; 1.0703x over previous
import functools

import jax
import jax.numpy as jnp
from jax import lax
from jax.experimental import pallas as pl
from jax.experimental.pallas import tpu as pltpu
from jax.experimental.pallas import tpu_sc as plsc

N = 10000
E = 160000
NC, NS = 2, 16
NP = N
EP = E
B = 100
RPT = NP // NS
EPT = EP // NS
DEGC = 16

f32 = jnp.float32
i32 = jnp.int32


@functools.lru_cache(maxsize=None)
def _make_sc_agg(C, edge_split=False):
    P = 1 if edge_split else C // NC
    n_out = NC if edge_split else C
    mesh = plsc.VectorSubcoreMesh(
        core_axis_name="c", subcore_axis_name="s",
        num_cores=NC, num_subcores=NS)
    bat, SL, DEPTH = 100, 50, 3
    G = DEPTH - 1
    nbatch = (EPT // NC if edge_split else EPT) // bat
    NSTG = nbatch // SL
    outs = [jax.ShapeDtypeStruct((NP, 128), f32) for _ in range(n_out)]
    scratch = [
        pltpu.VMEM_SHARED((NP, 128), f32),
        pltpu.VMEM((SL, bat), i32),
        pltpu.VMEM((SL, bat), i32),
    ] + [pltpu.VMEM((bat, 128), f32) for _ in range(DEPTH)] \
      + [pltpu.SemaphoreType.DMA] * (2 * DEPTH)

    def body(*refs):
        xs = refs[:C]
        srcr, dstr, zrow = refs[C:C + 3]
        outs_r = refs[C + 3:C + 3 + n_out]
        rest = refs[C + 3 + n_out:]
        acc, srcall, dstall = rest[:3]
        rows = rest[3:3 + DEPTH]
        semG = rest[3 + DEPTH:3 + 2 * DEPTH]
        semS = rest[3 + 2 * DEPTH:3 + 3 * DEPTH]

        cid = lax.axis_index("c")
        sid = lax.axis_index("s")
        rs = pl.ds(sid * RPT, RPT)

        for p in range(P):
            pltpu.sync_copy(zrow, acc.at[rs])
            plsc.subcore_barrier()
            for c in range(NC):
                @pl.when(cid == c)
                def _(p=p, c=c):
                    if edge_split:
                        chunk, slot = 0, c
                        row0 = (c * NS + sid) * nbatch
                    else:
                        chunk = slot = c * P + p
                        row0 = sid * nbatch
                    x = xs[chunk]

                    def start_g(i, k):
                        pltpu.async_copy(x.at[srcall.at[i]], rows[k],
                                         semG[k])

                    def wait_g(k):
                        pltpu.make_async_copy(x.at[srcall.at[0]], rows[k],
                                              semG[k]).wait()

                    def start_s(i, k):
                        pltpu.async_copy(rows[k], acc.at[dstall.at[i]],
                                         semS[k], add=True)

                    def wait_s(k):
                        pltpu.make_async_copy(rows[k], acc.at[dstall.at[0]],
                                              semS[k]).wait()

                    def stage(s, carry):
                        pltpu.sync_copy(
                            srcr.at[pl.ds(row0 + s * SL, SL)], srcall)
                        pltpu.sync_copy(
                            dstr.at[pl.ds(row0 + s * SL, SL)], dstall)
                        for j in range(G):
                            start_g(j, j)
                        for i in range(SL):
                            wait_g(i % DEPTH)
                            start_s(i, i % DEPTH)
                            if i + G < SL:
                                kn = (i + G) % DEPTH
                                if i >= 1:
                                    wait_s(kn)
                                start_g(i + G, kn)
                        for j in range(SL - G - 1, SL):
                            wait_s(j % DEPTH)
                        return carry

                    lax.fori_loop(0, NSTG, stage, 0)
            plsc.subcore_barrier()
            for c in range(NC):
                @pl.when(cid == c)
                def _(p=p, c=c):
                    slot = c if edge_split else c * P + p
                    pltpu.sync_copy(acc.at[rs], outs_r[slot].at[rs])

    return pl.kernel(body, out_type=outs, mesh=mesh, scratch_types=scratch,
                     compiler_params=pltpu.CompilerParams(
                         use_tc_tiling_on_sc=False),
                     name=f"sc_agg_c{C}" + ("_es" if edge_split else ""))


@functools.lru_cache(maxsize=None)
def _make_sc_deg():
    mesh = plsc.VectorSubcoreMesh(
        core_axis_name="c", subcore_axis_name="s",
        num_cores=NC, num_subcores=NS)
    nbatch = EPT // NC // B
    outs = [jax.ShapeDtypeStruct((NP, DEGC), f32) for _ in range(NC)]
    scratch = [
        pltpu.VMEM_SHARED((NP, DEGC), f32),
        pltpu.VMEM((nbatch, B), i32),
        pltpu.VMEM((B, DEGC), f32),
    ]

    def body(dstr, z16, ones_h, out0, out1, dacc, dstall, onesv):
        cid = lax.axis_index("c")
        sid = lax.axis_index("s")
        rs = pl.ds(sid * RPT, RPT)
        pltpu.sync_copy(z16, dacc.at[rs])
        pltpu.sync_copy(ones_h, onesv)
        for c in range(NC):
            @pl.when(cid == c)
            def _(c=c):
                row0 = (c * NS + sid) * nbatch
                pltpu.sync_copy(dstr.at[pl.ds(row0, nbatch)], dstall)
        plsc.subcore_barrier()

        def step(i, carry):
            pltpu.sync_copy(onesv, dacc.at[dstall.at[i]], add=True)
            return carry

        lax.fori_loop(0, nbatch, step, 0)
        plsc.subcore_barrier()
        outs_r = (out0, out1)
        for c in range(NC):
            @pl.when(cid == c)
            def _(c=c):
                pltpu.sync_copy(dacc.at[rs], outs_r[c].at[rs])

    return pl.kernel(body, out_type=outs, mesh=mesh, scratch_types=scratch,
                     compiler_params=pltpu.CompilerParams(
                         use_tc_tiling_on_sc=False),
                     name="sc_deg")


def _row_spec(BN):
    return pl.BlockSpec((BN, 128), lambda n: (n, 0))


def _make_tc_layer(C_in, C_out, relu, BN=200):
    def body(*refs):
        aggs = refs[:C_in]
        xs = refs[C_in:2 * C_in]
        inv, w, b = refs[2 * C_in:2 * C_in + 3]
        outs = refs[2 * C_in + 3:]
        h = jnp.concatenate(
            [(aggs[c][...] + xs[c][...]) * inv[...] for c in range(C_in)],
            axis=1)
        z = jnp.dot(h, w[...], preferred_element_type=f32) + b[...]
        if relu:
            z = jnp.maximum(z, 0.0)
        for co in range(C_out):
            outs[co][...] = z[:, co * 128:(co + 1) * 128]

    return pl.pallas_call(
        body,
        grid=(NP // BN,),
        in_specs=[_row_spec(BN)] * (2 * C_in) + [
            pl.BlockSpec((BN, 1), lambda n: (n, 0)),
            pl.BlockSpec((C_in * 128, C_out * 128), lambda n: (0, 0)),
            pl.BlockSpec((1, C_out * 128), lambda n: (0, 0)),
        ],
        out_specs=[_row_spec(BN)] * C_out,
        out_shape=[jax.ShapeDtypeStruct((NP, 128), f32)] * C_out,
    )


def _make_tc_layer_mm(C_in, C_out, BN=200):
    def body(*refs):
        aggs = refs[:C_in]
        xs = refs[C_in:2 * C_in]
        inv, w, b, w2 = refs[2 * C_in:2 * C_in + 4]
        outs = refs[2 * C_in + 4:]
        h = jnp.concatenate(
            [(aggs[c][...] + xs[c][...]) * inv[...] for c in range(C_in)],
            axis=1)
        z = jnp.maximum(
            jnp.dot(h, w[...], preferred_element_type=f32) + b[...], 0.0)
        outs[0][...] = jnp.dot(z, w2[...], preferred_element_type=f32)

    return pl.pallas_call(
        body,
        grid=(NP // BN,),
        in_specs=[_row_spec(BN)] * (2 * C_in) + [
            pl.BlockSpec((BN, 1), lambda n: (n, 0)),
            pl.BlockSpec((C_in * 128, C_out * 128), lambda n: (0, 0)),
            pl.BlockSpec((1, C_out * 128), lambda n: (0, 0)),
            pl.BlockSpec((C_out * 128, 128), lambda n: (0, 0)),
        ],
        out_specs=[_row_spec(BN)],
        out_shape=[jax.ShapeDtypeStruct((NP, 128), f32)],
    )


def _make_tc_combine(BN=200):
    def body(p0, p1, y, inv, b, out):
        out[...] = (p0[...] + p1[...] + y[...]) * inv[...] + b[...]

    return pl.pallas_call(
        body,
        grid=(NP // BN,),
        in_specs=[_row_spec(BN)] * 3 + [
            pl.BlockSpec((BN, 1), lambda n: (n, 0)),
            pl.BlockSpec((1, 128), lambda n: (0, 0)),
        ],
        out_specs=_row_spec(BN),
        out_shape=jax.ShapeDtypeStruct((NP, 128), f32),
    )


_tc_l0 = _make_tc_layer(2, 4, relu=True)
_tc_mid = _make_tc_layer(4, 4, relu=True)
_tc_l3mm = _make_tc_layer_mm(4, 4)
_tc_combine = _make_tc_combine()


def kernel(feat, edge_index, W0, b0, W1, b1, W2, b2, W3, b3, W4, b4):
    _sc_deg = _make_sc_deg()
    _sc_agg_l0 = _make_sc_agg(2)
    _sc_agg_mid = _make_sc_agg(4)
    _sc_agg_last = _make_sc_agg(1, edge_split=True)

    srcp = edge_index[0].astype(i32).reshape(EP // B, B)
    dstp = edge_index[1].astype(i32).reshape(EP // B, B)

    xc = [feat[:, 0:128], feat[:, 128:256]]

    z128 = jnp.zeros((RPT, 128), f32)
    z16 = jnp.zeros((RPT, DEGC), f32)
    ones = jnp.ones((B, DEGC), f32)

    d0, d1 = _sc_deg(dstp, z16, ones)
    inv = (1.0 / (d0[:, 0] + d1[:, 0] + 1.0)).reshape(NP, 1)

    agg0 = _sc_agg_l0(*xc, srcp, dstp, z128)

    xc = _tc_l0(*agg0, *xc, inv, W0, b0.reshape(1, 512))
    for W, b in ((W1, b1), (W2, b2)):
        aggs = _sc_agg_mid(*xc, srcp, dstp, z128)
        xc = _tc_mid(*aggs, *xc, inv, W, b.reshape(1, 512))

    aggs = _sc_agg_mid(*xc, srcp, dstp, z128)
    (y,) = _tc_l3mm(*aggs, *xc, inv, W3, b3.reshape(1, 512), W4)
    p0, p1 = _sc_agg_last(y, srcp, dstp, z128)
    return _tc_combine(p0, p1, y, inv, b4.reshape(1, 128))
